# Initial kernel scaffold; baseline (speedup 1.0000x reference)
import functools
import math

import jax
import jax.numpy as jnp
from jax import lax
from jax.experimental import pallas as pl
from jax.experimental.pallas import tpu as pltpu
from jax.experimental.pallas import tpu_sc as plsc

_B, _L, _H, _E = 1, 2048, 12, 64
_S, _D = 2048, 64
_LB = 256
_RB = 256
_NLB = _L // _LB
_NRB = _L // _RB


def _attn_body(q_ref, k_ref, v_ref, o_ref, m_ref):
    q = q_ref[0]
    k = k_ref[0]
    v = v_ref[0]
    s = lax.dot_general(q, k, (((1,), (1,)), ((), ())),
                        preferred_element_type=jnp.float32)
    rowmax = jnp.max(s, axis=1, keepdims=True)
    rowsum = jnp.sum(s, axis=1, keepdims=True)
    m_ref[0] = rowmax - rowsum * (1.0 / _S)
    p = jnp.exp((s - rowmax) * (1.0 / math.sqrt(_E)))
    denom = jnp.sum(p, axis=1, keepdims=True)
    o = lax.dot_general(p, v, (((1,), (0,)), ((), ())),
                        preferred_element_type=jnp.float32)
    o_ref[0] = o / denom


def _rank_body(mrow_ref, mcol_ref, idx_ref):
    h = pl.program_id(0)
    r = pl.program_id(1)
    row = mrow_ref[0]
    col = mcol_ref[0]
    gt = row > col
    eq = row == col
    j = lax.broadcasted_iota(jnp.int32, (_RB, _L), 1)
    i = lax.broadcasted_iota(jnp.int32, (_RB, _L), 0) + r * _RB
    wins = jnp.where(gt | (eq & (j < i)), 1, 0)
    rank = jnp.sum(wins, axis=1)
    idx_ref[0] = (rank * _H + h)[None, :]


_attn_call = pl.pallas_call(
    _attn_body,
    grid=(_H, _NLB),
    in_specs=[
        pl.BlockSpec((1, _LB, _E), lambda h, i: (h, i, 0)),
        pl.BlockSpec((1, _S, _E), lambda h, i: (h, 0, 0)),
        pl.BlockSpec((1, _S, _D), lambda h, i: (h, 0, 0)),
    ],
    out_specs=[
        pl.BlockSpec((1, _LB, _D), lambda h, i: (h, i, 0)),
        pl.BlockSpec((1, _LB, 1), lambda h, i: (h * _NLB + i, 0, 0)),
    ],
    out_shape=[
        jax.ShapeDtypeStruct((_H, _L, _D), jnp.float32),
        jax.ShapeDtypeStruct((_H * _NLB, _LB, 1), jnp.float32),
    ],
)

_rank_call = pl.pallas_call(
    _rank_body,
    grid=(_H, _NRB),
    in_specs=[
        pl.BlockSpec((1, 1, _L), lambda h, r: (h, 0, 0)),
        pl.BlockSpec((1, _RB, 1), lambda h, r: (h, r, 0)),
    ],
    out_specs=pl.BlockSpec((1, 1, _RB), lambda h, r: (h * _NRB + r, 0, 0)),
    out_shape=jax.ShapeDtypeStruct((_H * _NRB, 1, _RB), jnp.int32),
)

_NROWS = _H * _L
_IDX_MINOR = 128


def _make_scatter():
    info = plsc.get_sparse_core_info()
    nw = info.num_cores * info.num_subcores
    rows_per_w = _NROWS // nw
    chunks = rows_per_w // _IDX_MINOR

    @functools.partial(
        pl.kernel,
        out_type=jax.ShapeDtypeStruct((_NROWS, _D), jnp.float32),
        mesh=plsc.VectorSubcoreMesh(core_axis_name="c", subcore_axis_name="s"),
        scratch_types=[
            pltpu.VMEM((chunks, _IDX_MINOR), jnp.int32),
            pltpu.VMEM((rows_per_w, _D), jnp.float32),
            pltpu.SemaphoreType.DMA,
        ],
    )
    def scatter(o_hbm, idx_hbm, out_hbm, idx_v, rows_v, sem):
        wid = lax.axis_index("s") * info.num_cores + lax.axis_index("c")
        base = wid * rows_per_w
        pltpu.sync_copy(idx_hbm.at[wid], idx_v)
        pltpu.sync_copy(o_hbm.at[pl.ds(base, rows_per_w)], rows_v)
        for j in range(chunks):
            pltpu.async_copy(rows_v.at[pl.ds(j * _IDX_MINOR, _IDX_MINOR)],
                             out_hbm.at[idx_v.at[j]], sem).wait()

    return scatter, nw, chunks


def kernel(queries, keys, values):
    qh = jnp.transpose(queries[0], (1, 0, 2))
    kh = jnp.transpose(keys[0], (1, 0, 2))
    vh = jnp.transpose(values[0], (1, 0, 2))

    o, msp3 = _attn_call(qh, kh, vh)
    msp_row = msp3.reshape(_H, 1, _L)
    msp_col = msp3.reshape(_H, _L, 1)
    idx3 = _rank_call(msp_row, msp_col)

    scatter, nw, chunks = _make_scatter()
    idx_sc = idx3.reshape(nw, chunks, _IDX_MINOR)
    out_flat = scatter(o.reshape(_NROWS, _D), idx_sc)
    return out_flat.reshape(_B, _L, _H, _D)


# speedup vs baseline: 1.8277x; 1.8277x over previous
import functools
import math

import jax
import jax.numpy as jnp
from jax import lax
from jax.experimental import pallas as pl
from jax.experimental.pallas import tpu as pltpu
from jax.experimental.pallas import tpu_sc as plsc

_B, _L, _H, _E = 1, 2048, 12, 64
_S, _D = 2048, 64
_LB = 256
_RB = 256
_NLB = _L // _LB
_NRB = _L // _RB


def _attn_body(q_ref, k_ref, v_ref, o_ref, m_ref):
    q = q_ref[0]
    k = k_ref[0]
    v = v_ref[0]
    st = lax.dot_general(k, q, (((1,), (1,)), ((), ())),
                         preferred_element_type=jnp.float32)
    acc = jnp.zeros((8, _LB), jnp.float32)
    for t in range(_S // 8):
        acc = acc + st[t * 8:(t + 1) * 8, :]
    x = acc[0:4] + acc[4:8]
    x = x[0:2] + x[2:4]
    rowsum = x[0:1] + x[1:2]
    rowmax = jnp.max(st, axis=0, keepdims=True)
    m_ref[0] = rowmax - rowsum * (1.0 / _S)
    p = jnp.exp((st - rowmax) * (1.0 / math.sqrt(_E)))
    denom = jnp.sum(p, axis=0, keepdims=True)
    p = p * (1.0 / denom)
    o = lax.dot_general(p, v, (((0,), (0,)), ((), ())),
                        preferred_element_type=jnp.float32)
    o_ref[0] = o


def _rank_body(mrow_ref, mcol_ref, idx_ref):
    h = pl.program_id(0)
    r = pl.program_id(1)
    row = mrow_ref[0]
    col = mcol_ref[0]
    gt = row > col
    eq = row == col
    j = lax.broadcasted_iota(jnp.int32, (_RB, _L), 1)
    i = lax.broadcasted_iota(jnp.int32, (_RB, _L), 0) + r * _RB
    wins = jnp.where(gt | (eq & (j < i)), 1, 0)
    rank = jnp.sum(wins, axis=1)
    idx_ref[0] = (rank * _H + h)[None, :]


_attn_call = pl.pallas_call(
    _attn_body,
    grid=(_H, _NLB),
    in_specs=[
        pl.BlockSpec((1, _LB, _E), lambda h, i: (h, i, 0)),
        pl.BlockSpec((1, _S, _E), lambda h, i: (h, 0, 0)),
        pl.BlockSpec((1, _S, _D), lambda h, i: (h, 0, 0)),
    ],
    out_specs=[
        pl.BlockSpec((1, _LB, _D), lambda h, i: (h, i, 0)),
        pl.BlockSpec((1, 1, _LB), lambda h, i: (h * _NLB + i, 0, 0)),
    ],
    out_shape=[
        jax.ShapeDtypeStruct((_H, _L, _D), jnp.float32),
        jax.ShapeDtypeStruct((_H * _NLB, 1, _LB), jnp.float32),
    ],
)

_rank_call = pl.pallas_call(
    _rank_body,
    grid=(_H, _NRB),
    in_specs=[
        pl.BlockSpec((1, 1, _L), lambda h, r: (h, 0, 0)),
        pl.BlockSpec((1, _RB, 1), lambda h, r: (h, r, 0)),
    ],
    out_specs=pl.BlockSpec((1, 1, _RB), lambda h, r: (h * _NRB + r, 0, 0)),
    out_shape=jax.ShapeDtypeStruct((_H * _NRB, 1, _RB), jnp.int32),
)

_NROWS = _H * _L
_IDX_MINOR = 128


def _make_scatter():
    info = plsc.get_sparse_core_info()
    nw = info.num_cores * info.num_subcores
    rows_per_w = _NROWS // nw
    chunks = rows_per_w // _IDX_MINOR

    @functools.partial(
        pl.kernel,
        out_type=jax.ShapeDtypeStruct((_NROWS, _D), jnp.float32),
        mesh=plsc.VectorSubcoreMesh(core_axis_name="c", subcore_axis_name="s"),
        scratch_types=[
            pltpu.VMEM((chunks, _IDX_MINOR), jnp.int32),
            pltpu.VMEM((rows_per_w, _D), jnp.float32),
            pltpu.SemaphoreType.DMA,
        ],
        compiler_params=pltpu.CompilerParams(use_tc_tiling_on_sc=False),
    )
    def scatter(o_hbm, idx_hbm, out_hbm, idx_v, rows_v, sem):
        wid = lax.axis_index("s") * info.num_cores + lax.axis_index("c")
        base = wid * rows_per_w
        pltpu.sync_copy(idx_hbm.at[wid], idx_v)
        pltpu.sync_copy(o_hbm.at[pl.ds(base, rows_per_w)], rows_v)
        for j in range(chunks):
            pltpu.async_copy(rows_v.at[pl.ds(j * _IDX_MINOR, _IDX_MINOR)],
                             out_hbm.at[idx_v.at[j]], sem).wait()

    return scatter, nw, chunks


def kernel(queries, keys, values):
    qh = jnp.transpose(queries[0], (1, 0, 2))
    kh = jnp.transpose(keys[0], (1, 0, 2))
    vh = jnp.transpose(values[0], (1, 0, 2))

    o, msp3 = _attn_call(qh, kh, vh)
    msp_row = msp3.reshape(_H, 1, _L)
    msp_col = msp3.reshape(_H, _L, 1)
    idx3 = _rank_call(msp_row, msp_col)

    scatter, nw, chunks = _make_scatter()
    idx_sc = idx3.reshape(nw, chunks, _IDX_MINOR)
    out_flat = scatter(o.reshape(_NROWS, _D), idx_sc)
    return out_flat.reshape(_B, _L, _H, _D)
